# baseline (speedup 1.0000x reference)
import functools

import jax
import jax.numpy as jnp
from jax.experimental import pallas as pl
from jax.experimental.pallas import tpu as pltpu

_K = 1024
_D = 64
_B = 64
_HW = 576
_N = _B * _HW
_COMMIT = 0.25


def _vq_block(x_ref, e_ref, et_ref, q_ref, loss_ref, ppl_ref, cnt_ref, sse_ref):
    b = pl.program_id(0)

    @pl.when(b == 0)
    def _init():
        cnt_ref[...] = jnp.zeros_like(cnt_ref)
        sse_ref[...] = jnp.zeros_like(sse_ref)

    x = x_ref[0]
    e = e_ref[...]
    enorm = jnp.sum(e * e, axis=1)
    fnorm = jnp.sum(x * x, axis=0)
    c = jax.lax.dot_general(x, e, (((0,), (1,)), ((), ())),
                            preferred_element_type=jnp.float32)
    s = fnorm[:, None] + enorm[None, :]
    dist = s - 2.0 * c
    minv = jnp.min(dist, axis=1)
    nkiota = jax.lax.broadcasted_iota(jnp.int32, (_HW, _K), 1)
    idx = jnp.min(jnp.where(dist == minv[:, None], nkiota, _K), axis=1)
    one_hot = (jax.lax.broadcasted_iota(jnp.int32, (_K, _HW), 0)
               == idx[None, :]).astype(jnp.float32)
    q = jax.lax.dot_general(et_ref[...], one_hot, (((1,), (0,)), ((), ())),
                            preferred_element_type=jnp.float32)
    diff = q - x
    q_ref[0] = x + diff
    sse_ref[...] = sse_ref[...] + jnp.sum(diff * diff)
    cnt_ref[...] = cnt_ref[...] + jnp.sum(one_hot, axis=1)

    @pl.when(b == _B - 1)
    def _finalize():
        m = sse_ref[...] / jnp.float32(_N * _D)
        loss_ref[...] = m + _COMMIT * m
        p = cnt_ref[...] / jnp.float32(_N)
        ent = jnp.sum(p * jnp.log(p + 1e-10))
        ppl_ref[...] = jnp.exp(-ent) * jnp.ones_like(ppl_ref)


@functools.partial(jax.jit)
def kernel(inputs, embedding_weight):
    x2 = inputs.reshape(_B, _D, _HW)
    et = embedding_weight.T
    q2, loss, ppl = pl.pallas_call(
        _vq_block,
        grid=(_B,),
        in_specs=[
            pl.BlockSpec((1, _D, _HW), lambda b: (b, 0, 0)),
            pl.BlockSpec((_K, _D), lambda b: (0, 0)),
            pl.BlockSpec((_D, _K), lambda b: (0, 0)),
        ],
        out_specs=[
            pl.BlockSpec((1, _D, _HW), lambda b: (b, 0, 0)),
            pl.BlockSpec((1, 1), lambda b: (0, 0)),
            pl.BlockSpec((1, 1), lambda b: (0, 0)),
        ],
        out_shape=[
            jax.ShapeDtypeStruct((_B, _D, _HW), jnp.float32),
            jax.ShapeDtypeStruct((1, 1), jnp.float32),
            jax.ShapeDtypeStruct((1, 1), jnp.float32),
        ],
        scratch_shapes=[
            pltpu.VMEM((_K,), jnp.float32),
            pltpu.VMEM((1, 1), jnp.float32),
        ],
    )(x2, embedding_weight, et)
    return loss[0, 0], q2.reshape(_B, _D, 24, 24), ppl[0, 0]
